# baseline (speedup 1.0000x reference)
import functools

import jax
import jax.numpy as jnp
from jax import lax
from jax.experimental import pallas as pl
from jax.experimental.pallas import tpu as pltpu
from jax.experimental.pallas import tpu_sc as plsc

N = 10000
NP = 10240
E = 320000
D_IN = 128
D_EDGE = 16
HID = 32

NC = 2
NS = 16
NW = NC * NS
E_PER_W = E // NW
CHUNK = 2000
NCH = E_PER_W // CHUNK
SLICE = NP // NS

_mesh = plsc.VectorSubcoreMesh(core_axis_name="c", subcore_axis_name="s")


def _fill_1d(buf, n, val):
    def body(i, carry):
        buf[pl.ds(i * 16, 16)] = jnp.full((16,), val, jnp.float32)
        return carry
    lax.fori_loop(0, n // 16, body, 0)


DEG_CH = (E // NS) // CHUNK


@functools.partial(
    pl.kernel,
    mesh=_mesh,
    compiler_params=pltpu.CompilerParams(use_tc_tiling_on_sc=False,
                                         needs_layout_passes=False),
    out_type=[
        jax.ShapeDtypeStruct((NC, NP, HID), jnp.float32),
        jax.ShapeDtypeStruct((NP, HID), jnp.float32),
        jax.ShapeDtypeStruct((NP, HID), jnp.float32),
        jax.ShapeDtypeStruct((NC, NP), jnp.float32),
    ],
    scratch_types=[
        pltpu.VMEM((CHUNK,), jnp.int32),
        pltpu.VMEM((CHUNK,), jnp.int32),
        pltpu.VMEM((CHUNK,), jnp.float32),
        pltpu.VMEM((CHUNK, HID), jnp.float32),
        pltpu.VMEM((SLICE,), jnp.float32),
        pltpu.VMEM((SLICE, HID), jnp.float32),
        pltpu.VMEM_SHARED((NP,), jnp.float32),
        pltpu.VMEM_SHARED((NP, HID), jnp.float32),
        pltpu.SemaphoreType.DMA,
    ],
)
def _mega_sc(xenc_hbm, ei_hbm,
             s_hbm, y0_hbm, y1_hbm, dinv_hbm,
             ridx_v, cidx_v, ones_v, rows_v, dbuf_v, ybuf_v,
             sh_deg, sh_s, sem):
    c = lax.axis_index("c")
    s = lax.axis_index("s")
    wid = s * NC + c
    _fill_1d(ones_v, CHUNK, 1.0)
    _fill_1d(dbuf_v, SLICE, 0.0)

    def zbody(i, carry):
        ybuf_v[i, pl.ds(0, 16)] = jnp.zeros((16,), jnp.float32)
        ybuf_v[i, pl.ds(16, 16)] = jnp.zeros((16,), jnp.float32)
        return carry
    lax.fori_loop(0, SLICE, zbody, 0)
    pltpu.sync_copy(dbuf_v, sh_deg.at[pl.ds(s * SLICE, SLICE)])
    pltpu.sync_copy(ybuf_v, sh_s.at[pl.ds(s * SLICE, SLICE), :])
    plsc.subcore_barrier()
    for k in range(DEG_CH):
        base = s * (E // NS) + k * CHUNK
        pltpu.sync_copy(ei_hbm.at[1, pl.ds(base, CHUNK)], cidx_v)
        pltpu.sync_copy(ones_v, sh_deg.at[cidx_v], add=True)
    plsc.subcore_barrier()
    pltpu.sync_copy(sh_deg.at[pl.ds(s * SLICE, SLICE)], dbuf_v)

    def nr(i, carry):
        d = dbuf_v[pl.ds(i * 16, 16)] + 1.0
        xi = plsc.bitcast(d, jnp.int32)
        xi = jnp.full((16,), 0x5F3759DF, jnp.int32) \
            - lax.shift_right_logical(xi, jnp.ones((16,), jnp.int32))
        xx = plsc.bitcast(xi, jnp.float32)
        for _ in range(4):
            xx = xx * (1.5 - 0.5 * d * xx * xx)
        dbuf_v[pl.ds(i * 16, 16)] = xx
        return carry
    lax.fori_loop(0, SLICE // 16, nr, 0)
    pltpu.sync_copy(dbuf_v, dinv_hbm.at[c, pl.ds(s * SLICE, SLICE)])
    pltpu.sync_copy(xenc_hbm.at[pl.ds(s * SLICE, SLICE), :], ybuf_v)

    def scale_row(i, carry):
        splat = jnp.zeros((16,), jnp.int32) + i
        dv = plsc.load_gather(dbuf_v, [splat])
        ybuf_v[i, pl.ds(0, 16)] = ybuf_v[i, pl.ds(0, 16)] * dv
        ybuf_v[i, pl.ds(16, 16)] = ybuf_v[i, pl.ds(16, 16)] * dv
        return carry
    lax.fori_loop(0, SLICE, scale_row, 0)

    @pl.when(c == 0)
    def _():
        pltpu.sync_copy(ybuf_v, y0_hbm.at[pl.ds(s * SLICE, SLICE), :])

    @pl.when(c == 1)
    def _():
        pltpu.sync_copy(ybuf_v, y1_hbm.at[pl.ds(s * SLICE, SLICE), :])
    plsc.subcore_barrier()
    for k in range(NCH):
        base = wid * E_PER_W + k * CHUNK
        pltpu.sync_copy(ei_hbm.at[0, pl.ds(base, CHUNK)], ridx_v)
        pltpu.sync_copy(ei_hbm.at[1, pl.ds(base, CHUNK)], cidx_v)

        @pl.when(c == 0)
        def _():
            pltpu.async_copy(y0_hbm.at[ridx_v], rows_v, sem).wait()

        @pl.when(c == 1)
        def _():
            pltpu.async_copy(y1_hbm.at[ridx_v], rows_v, sem).wait()
        pltpu.sync_copy(rows_v, sh_s.at[cidx_v], add=True)
    plsc.subcore_barrier()
    pltpu.sync_copy(sh_s.at[pl.ds(s * SLICE, SLICE), :], ybuf_v)
    pltpu.sync_copy(ybuf_v, s_hbm.at[c, pl.ds(s * SLICE, SLICE), :])


@functools.partial(
    pl.kernel,
    mesh=_mesh,
    compiler_params=pltpu.CompilerParams(use_tc_tiling_on_sc=False,
                                         needs_layout_passes=False),
    out_type=jax.ShapeDtypeStruct((E,), jnp.float32),
    scratch_types=[
        pltpu.VMEM((NP,), jnp.float32),
        pltpu.VMEM((NP,), jnp.float32),
        pltpu.VMEM((CHUNK,), jnp.int32),
        pltpu.VMEM((CHUNK,), jnp.int32),
        pltpu.VMEM((CHUNK,), jnp.float32),
        pltpu.VMEM((CHUNK,), jnp.float32),
    ],
)
def _edgeout_sc(u_hbm, v_hbm, eb_hbm, ei_hbm, out_hbm,
                u_v, v_v, ridx_v, cidx_v, eb_v, o_v):
    c = lax.axis_index("c")
    s = lax.axis_index("s")
    wid = s * NC + c
    pltpu.sync_copy(u_hbm, u_v)
    pltpu.sync_copy(v_hbm, v_v)
    for k in range(NCH):
        base = wid * E_PER_W + k * CHUNK
        pltpu.sync_copy(ei_hbm.at[0, pl.ds(base, CHUNK)], ridx_v)
        pltpu.sync_copy(ei_hbm.at[1, pl.ds(base, CHUNK)], cidx_v)
        pltpu.sync_copy(eb_hbm.at[pl.ds(base, CHUNK)], eb_v)

        def body(j, carry):
            r = ridx_v[pl.ds(j * 16, 16)]
            cc = cidx_v[pl.ds(j * 16, 16)]
            g = (plsc.load_gather(u_v, [r])
                 + plsc.load_gather(v_v, [cc])
                 + eb_v[pl.ds(j * 16, 16)])
            o_v[pl.ds(j * 16, 16)] = g
            return carry
        lax.fori_loop(0, CHUNK // 16, body, 0)
        pltpu.sync_copy(o_v, out_hbm.at[pl.ds(base, CHUNK)])


_BE = 6400


def _pre_body(x_ref, wne_ref, bne_ref, ea_ref, wee_ref, bee_ref, wout_ref,
              bout_ref, xenc_ref, eb_ref):
    @pl.when(pl.program_id(0) == 0)
    def _():
        xw = jnp.dot(x_ref[...], wne_ref[...],
                     preferred_element_type=jnp.float32)
        xenc_ref[...] = jnp.maximum(xw + bne_ref[...], 0.0)

    t = jnp.dot(ea_ref[...], wee_ref[...], preferred_element_type=jnp.float32)
    t = jnp.maximum(t + bee_ref[...], 0.0)
    w3 = wout_ref[2 * HID:3 * HID, :]
    t3 = (jnp.dot(t, w3, preferred_element_type=jnp.float32)
          + bout_ref[...])
    i = pl.program_id(0)
    eb_ref[pl.ds(i * _BE, _BE)] = jnp.reshape(t3, (_BE,))


def _pre_tc(x_pad, W_ne, b_ne2, edge_attr, W_ee, b_ee2, W_out, b_out2):
    return pl.pallas_call(
        _pre_body,
        grid=(E // _BE,),
        in_specs=[
            pl.BlockSpec((NP, D_IN), lambda i: (0, 0)),
            pl.BlockSpec((D_IN, HID), lambda i: (0, 0)),
            pl.BlockSpec((1, HID), lambda i: (0, 0)),
            pl.BlockSpec((_BE, D_EDGE), lambda i: (i, 0)),
            pl.BlockSpec((D_EDGE, HID), lambda i: (0, 0)),
            pl.BlockSpec((1, HID), lambda i: (0, 0)),
            pl.BlockSpec((3 * HID, 1), lambda i: (0, 0)),
            pl.BlockSpec((1, 1), lambda i: (0, 0)),
        ],
        out_specs=[
            pl.BlockSpec((NP, HID), lambda i: (0, 0)),
            pl.BlockSpec((E,), lambda i: (0,)),
        ],
        out_shape=[
            jax.ShapeDtypeStruct((NP, HID), jnp.float32),
            jax.ShapeDtypeStruct((E,), jnp.float32),
        ],
    )(x_pad, W_ne, b_ne2, edge_attr, W_ee, b_ee2, W_out, b_out2)


def _huv_body(s_ref, y_ref, dinv_ref, wz_ref, lzw_ref, lzb_ref, bz_ref,
              wh_ref, lhw_ref, lhb_ref, bh_ref, wout_ref, u_ref, v_ref):
    agg = (s_ref[0] + s_ref[1] + y_ref[...]) * dinv_ref[...]
    lzw = lzw_ref[0:HID, :]
    lhw = lhw_ref[0:HID, :]
    mz = jnp.dot(wz_ref[...], lzw, preferred_element_type=jnp.float32)
    cz = jnp.dot(bz_ref[...], lzw, preferred_element_type=jnp.float32) \
        + lzb_ref[...]
    mh = jnp.dot(wh_ref[...], lhw, preferred_element_type=jnp.float32)
    ch = jnp.dot(bh_ref[...], lhw, preferred_element_type=jnp.float32) \
        + lhb_ref[...]
    z = jax.nn.sigmoid(
        jnp.dot(agg, mz, preferred_element_type=jnp.float32) + cz)
    ht = jnp.tanh(jnp.dot(agg, mh, preferred_element_type=jnp.float32) + ch)
    h = (1.0 - z) * ht
    u_ref[...] = jnp.reshape(
        jnp.dot(h, wout_ref[0:HID, :], preferred_element_type=jnp.float32),
        (NP,))
    v_ref[...] = jnp.reshape(
        jnp.dot(h, wout_ref[HID:2 * HID, :],
                preferred_element_type=jnp.float32), (NP,))


def _huv_tc(S, y, dinv, Wz, LzW, Lzb2, bz2, Wh, LhW, Lhb2, bh2, W_out):
    return pl.pallas_call(
        _huv_body,
        out_shape=[
            jax.ShapeDtypeStruct((NP,), jnp.float32),
            jax.ShapeDtypeStruct((NP,), jnp.float32),
        ],
    )(S, y, dinv, Wz, LzW, Lzb2, bz2, Wh, LhW, Lhb2, bh2, W_out)


def kernel(x, edge_index, edge_attr, W_ne, b_ne, W_ee, b_ee, Wz, bz, LzW,
           Lzb, Wr, br, LrW, Lrb, Wh, bh, LhW, Lhb, W_out, b_out):
    ei = edge_index.astype(jnp.int32)
    x_pad = jnp.pad(x, ((0, NP - N), (0, 0)))

    x_enc, eb = _pre_tc(x_pad, W_ne, b_ne.reshape(1, HID), edge_attr,
                        W_ee, b_ee.reshape(1, HID), W_out,
                        b_out.reshape(1, 1))
    S, y0, _y1, dinvs = _mega_sc(x_enc, ei)
    u1, v1 = _huv_tc(S, y0, dinvs[0].reshape(NP, 1), Wz, LzW,
                     Lzb.reshape(1, HID), bz.reshape(1, HID), Wh, LhW,
                     Lhb.reshape(1, HID), bh.reshape(1, HID), W_out)
    out = _edgeout_sc(u1, v1, eb, ei)
    return out.reshape(E, 1)


# speedup vs baseline: 1.9754x; 1.9754x over previous
import functools

import jax
import jax.numpy as jnp
from jax import lax
from jax.experimental import pallas as pl
from jax.experimental.pallas import tpu as pltpu
from jax.experimental.pallas import tpu_sc as plsc

N = 10000
NP = 10240
E = 320000
D_IN = 128
D_EDGE = 16
HID = 32

NC = 2
NS = 16
NW = NC * NS
E_PER_W = E // NW
CHUNK = 2000
NCH = E_PER_W // CHUNK
SLICE = NP // NS

_mesh = plsc.VectorSubcoreMesh(core_axis_name="c", subcore_axis_name="s")


def _fill_1d(buf, n, val):
    def body(i, carry):
        buf[pl.ds(i * 16, 16)] = jnp.full((16,), val, jnp.float32)
        return carry
    lax.fori_loop(0, n // 16, body, 0)


DEG_CH = (E // NS) // CHUNK


@functools.partial(
    pl.kernel,
    mesh=_mesh,
    compiler_params=pltpu.CompilerParams(use_tc_tiling_on_sc=False,
                                         needs_layout_passes=False),
    out_type=[
        jax.ShapeDtypeStruct((NC, NP, HID), jnp.float32),
        jax.ShapeDtypeStruct((NP, HID), jnp.float32),
        jax.ShapeDtypeStruct((NP, HID), jnp.float32),
        jax.ShapeDtypeStruct((NC, NP), jnp.float32),
    ],
    scratch_types=[
        pltpu.VMEM((CHUNK,), jnp.int32),
        pltpu.VMEM((CHUNK,), jnp.int32),
        pltpu.VMEM((CHUNK,), jnp.float32),
        pltpu.VMEM((CHUNK, HID), jnp.float32),
        pltpu.VMEM((SLICE,), jnp.float32),
        pltpu.VMEM((SLICE, HID), jnp.float32),
        pltpu.VMEM_SHARED((NP,), jnp.float32),
        pltpu.VMEM_SHARED((NP, HID), jnp.float32),
        pltpu.SemaphoreType.DMA,
    ],
)
def _mega_sc(xenc_hbm, ei_hbm,
             s_hbm, y0_hbm, y1_hbm, dinv_hbm,
             ridx_v, cidx_v, ones_v, rows_v, dbuf_v, ybuf_v,
             sh_deg, sh_s, sem):
    c = lax.axis_index("c")
    s = lax.axis_index("s")
    wid = s * NC + c
    _fill_1d(ones_v, CHUNK, 1.0)
    _fill_1d(dbuf_v, SLICE, 0.0)

    def zbody(i, carry):
        ybuf_v[i, pl.ds(0, 16)] = jnp.zeros((16,), jnp.float32)
        ybuf_v[i, pl.ds(16, 16)] = jnp.zeros((16,), jnp.float32)
        return carry
    lax.fori_loop(0, SLICE, zbody, 0)
    pltpu.sync_copy(dbuf_v, sh_deg.at[pl.ds(s * SLICE, SLICE)])
    pltpu.sync_copy(ybuf_v, sh_s.at[pl.ds(s * SLICE, SLICE), :])
    plsc.subcore_barrier()
    for k in range(DEG_CH):
        base = s * (E // NS) + k * CHUNK
        pltpu.sync_copy(ei_hbm.at[1, pl.ds(base, CHUNK)], cidx_v)
        pltpu.sync_copy(ones_v, sh_deg.at[cidx_v], add=True)
    plsc.subcore_barrier()
    pltpu.sync_copy(sh_deg.at[pl.ds(s * SLICE, SLICE)], dbuf_v)

    def nr(i, carry):
        d = dbuf_v[pl.ds(i * 16, 16)] + 1.0
        xi = plsc.bitcast(d, jnp.int32)
        xi = jnp.full((16,), 0x5F3759DF, jnp.int32) \
            - lax.shift_right_logical(xi, jnp.ones((16,), jnp.int32))
        xx = plsc.bitcast(xi, jnp.float32)
        for _ in range(4):
            xx = xx * (1.5 - 0.5 * d * xx * xx)
        dbuf_v[pl.ds(i * 16, 16)] = xx
        return carry
    lax.fori_loop(0, SLICE // 16, nr, 0)
    pltpu.sync_copy(dbuf_v, dinv_hbm.at[c, pl.ds(s * SLICE, SLICE)])
    pltpu.sync_copy(xenc_hbm.at[pl.ds(s * SLICE, SLICE), :], ybuf_v)

    def scale_row(i, carry):
        splat = jnp.zeros((16,), jnp.int32) + i
        dv = plsc.load_gather(dbuf_v, [splat])
        ybuf_v[i, pl.ds(0, 16)] = ybuf_v[i, pl.ds(0, 16)] * dv
        ybuf_v[i, pl.ds(16, 16)] = ybuf_v[i, pl.ds(16, 16)] * dv
        return carry
    lax.fori_loop(0, SLICE, scale_row, 0)

    @pl.when(c == 0)
    def _():
        pltpu.sync_copy(ybuf_v, y0_hbm.at[pl.ds(s * SLICE, SLICE), :])

    @pl.when(c == 1)
    def _():
        pltpu.sync_copy(ybuf_v, y1_hbm.at[pl.ds(s * SLICE, SLICE), :])
    plsc.subcore_barrier()
    for k in range(NCH):
        base = wid * E_PER_W + k * CHUNK
        pltpu.sync_copy(ei_hbm.at[0, pl.ds(base, CHUNK)], ridx_v)
        pltpu.sync_copy(ei_hbm.at[1, pl.ds(base, CHUNK)], cidx_v)

        @pl.when(c == 0)
        def _():
            pltpu.async_copy(y0_hbm.at[ridx_v], rows_v, sem).wait()

        @pl.when(c == 1)
        def _():
            pltpu.async_copy(y1_hbm.at[ridx_v], rows_v, sem).wait()
        pltpu.sync_copy(rows_v, sh_s.at[cidx_v], add=True)
    plsc.subcore_barrier()
    pltpu.sync_copy(sh_s.at[pl.ds(s * SLICE, SLICE), :], ybuf_v)
    pltpu.sync_copy(ybuf_v, s_hbm.at[c, pl.ds(s * SLICE, SLICE), :])


@functools.partial(
    pl.kernel,
    mesh=_mesh,
    compiler_params=pltpu.CompilerParams(use_tc_tiling_on_sc=False,
                                         needs_layout_passes=False),
    out_type=jax.ShapeDtypeStruct((E,), jnp.float32),
    scratch_types=[
        pltpu.VMEM((NP,), jnp.float32),
        pltpu.VMEM((NP,), jnp.float32),
        pltpu.VMEM((CHUNK,), jnp.int32),
        pltpu.VMEM((CHUNK,), jnp.int32),
        pltpu.VMEM((CHUNK,), jnp.float32),
        pltpu.VMEM((CHUNK,), jnp.float32),
    ],
)
def _edgeout_sc(u_hbm, v_hbm, eb_hbm, ei_hbm, out_hbm,
                u_v, v_v, ridx_v, cidx_v, eb_v, o_v):
    c = lax.axis_index("c")
    s = lax.axis_index("s")
    wid = s * NC + c
    pltpu.sync_copy(u_hbm, u_v)
    pltpu.sync_copy(v_hbm, v_v)
    for k in range(NCH):
        base = wid * E_PER_W + k * CHUNK
        pltpu.sync_copy(ei_hbm.at[0, pl.ds(base, CHUNK)], ridx_v)
        pltpu.sync_copy(ei_hbm.at[1, pl.ds(base, CHUNK)], cidx_v)
        pltpu.sync_copy(eb_hbm.at[pl.ds(base, CHUNK)], eb_v)

        def body(j, carry):
            r = ridx_v[pl.ds(j * 16, 16)]
            cc = cidx_v[pl.ds(j * 16, 16)]
            g = (plsc.load_gather(u_v, [r])
                 + plsc.load_gather(v_v, [cc])
                 + eb_v[pl.ds(j * 16, 16)])
            o_v[pl.ds(j * 16, 16)] = g
            return carry
        lax.fori_loop(0, CHUNK // 16, body, 0)
        pltpu.sync_copy(o_v, out_hbm.at[pl.ds(base, CHUNK)])


_BE = 6400


def _pre_body(x_ref, wne_ref, bne_ref, eat_ref, wee_ref, bee_ref, wout_ref,
              bout_ref, xenc_ref, eb_ref):
    @pl.when(pl.program_id(0) == 0)
    def _():
        xw = jnp.dot(x_ref[...], wne_ref[...],
                     preferred_element_type=jnp.float32)
        xenc_ref[...] = jnp.maximum(xw + bne_ref[...], 0.0)

    tt = lax.dot_general(wee_ref[...], eat_ref[...],
                         (((0,), (0,)), ((), ())),
                         preferred_element_type=jnp.float32)
    tt = jnp.maximum(tt + bee_ref[...], 0.0)
    w3 = wout_ref[2 * HID:3 * HID, :]
    st = lax.dot_general(w3, tt, (((0,), (0,)), ((), ())),
                         preferred_element_type=jnp.float32) + bout_ref[...]
    i = pl.program_id(0)
    eb_ref[pl.ds(i * _BE, _BE)] = jnp.reshape(st, (_BE,))


def _pre_tc(x_pad, W_ne, b_ne2, ea_t, W_ee, b_ee_col, W_out, b_out2):
    return pl.pallas_call(
        _pre_body,
        grid=(E // _BE,),
        in_specs=[
            pl.BlockSpec((NP, D_IN), lambda i: (0, 0)),
            pl.BlockSpec((D_IN, HID), lambda i: (0, 0)),
            pl.BlockSpec((1, HID), lambda i: (0, 0)),
            pl.BlockSpec((D_EDGE, _BE), lambda i: (0, i)),
            pl.BlockSpec((D_EDGE, HID), lambda i: (0, 0)),
            pl.BlockSpec((HID, 1), lambda i: (0, 0)),
            pl.BlockSpec((3 * HID, 1), lambda i: (0, 0)),
            pl.BlockSpec((1, 1), lambda i: (0, 0)),
        ],
        out_specs=[
            pl.BlockSpec((NP, HID), lambda i: (0, 0)),
            pl.BlockSpec((E,), lambda i: (0,)),
        ],
        out_shape=[
            jax.ShapeDtypeStruct((NP, HID), jnp.float32),
            jax.ShapeDtypeStruct((E,), jnp.float32),
        ],
    )(x_pad, W_ne, b_ne2, ea_t, W_ee, b_ee_col, W_out, b_out2)


def _huv_body(s_ref, y_ref, dinv_ref, wz_ref, lzw_ref, lzb_ref, bz_ref,
              wh_ref, lhw_ref, lhb_ref, bh_ref, wout_ref, u_ref, v_ref):
    agg = (s_ref[0] + s_ref[1] + y_ref[...]) * dinv_ref[...]
    lzw = lzw_ref[0:HID, :]
    lhw = lhw_ref[0:HID, :]
    mz = jnp.dot(wz_ref[...], lzw, preferred_element_type=jnp.float32)
    cz = jnp.dot(bz_ref[...], lzw, preferred_element_type=jnp.float32) \
        + lzb_ref[...]
    mh = jnp.dot(wh_ref[...], lhw, preferred_element_type=jnp.float32)
    ch = jnp.dot(bh_ref[...], lhw, preferred_element_type=jnp.float32) \
        + lhb_ref[...]
    z = jax.nn.sigmoid(
        jnp.dot(agg, mz, preferred_element_type=jnp.float32) + cz)
    ht = jnp.tanh(jnp.dot(agg, mh, preferred_element_type=jnp.float32) + ch)
    h = (1.0 - z) * ht
    u_ref[...] = jnp.reshape(
        jnp.dot(h, wout_ref[0:HID, :], preferred_element_type=jnp.float32),
        (NP,))
    v_ref[...] = jnp.reshape(
        jnp.dot(h, wout_ref[HID:2 * HID, :],
                preferred_element_type=jnp.float32), (NP,))


def _huv_tc(S, y, dinv, Wz, LzW, Lzb2, bz2, Wh, LhW, Lhb2, bh2, W_out):
    return pl.pallas_call(
        _huv_body,
        out_shape=[
            jax.ShapeDtypeStruct((NP,), jnp.float32),
            jax.ShapeDtypeStruct((NP,), jnp.float32),
        ],
    )(S, y, dinv, Wz, LzW, Lzb2, bz2, Wh, LhW, Lhb2, bh2, W_out)


def kernel(x, edge_index, edge_attr, W_ne, b_ne, W_ee, b_ee, Wz, bz, LzW,
           Lzb, Wr, br, LrW, Lrb, Wh, bh, LhW, Lhb, W_out, b_out):
    ei = edge_index.astype(jnp.int32)
    x_pad = jnp.pad(x, ((0, NP - N), (0, 0)))

    x_enc, eb = _pre_tc(x_pad, W_ne, b_ne.reshape(1, HID), edge_attr.T,
                        W_ee, b_ee.reshape(HID, 1), W_out,
                        b_out.reshape(1, 1))
    S, y0, _y1, dinvs = _mega_sc(x_enc, ei)
    u1, v1 = _huv_tc(S, y0, dinvs[0].reshape(NP, 1), Wz, LzW,
                     Lzb.reshape(1, HID), bz.reshape(1, HID), Wh, LhW,
                     Lhb.reshape(1, HID), bh.reshape(1, HID), W_out)
    out = _edgeout_sc(u1, v1, eb, ei)
    return out.reshape(E, 1)


# speedup vs baseline: 2.2262x; 1.1269x over previous
import functools

import jax
import jax.numpy as jnp
from jax import lax
from jax.experimental import pallas as pl
from jax.experimental.pallas import tpu as pltpu
from jax.experimental.pallas import tpu_sc as plsc

N = 10000
NP = 10240
E = 320000
D_IN = 128
D_EDGE = 16
HID = 32

NC = 2
NS = 16
NW = NC * NS
E_PER_W = E // NW
CHUNK = 2000
NCH = E_PER_W // CHUNK
SLICE = NP // NS

_mesh = plsc.VectorSubcoreMesh(core_axis_name="c", subcore_axis_name="s")


def _fill_1d(buf, n, val):
    def body(i, carry):
        buf[pl.ds(i * 16, 16)] = jnp.full((16,), val, jnp.float32)
        return carry
    lax.fori_loop(0, n // 16, body, 0)


CH = 1000
NCH_C = E_PER_W // CH
DEG_CH = (E // NS) // CH


@functools.partial(
    pl.kernel,
    mesh=_mesh,
    compiler_params=pltpu.CompilerParams(use_tc_tiling_on_sc=False,
                                         needs_layout_passes=False),
    out_type=[
        jax.ShapeDtypeStruct((NC, NP, HID), jnp.float32),
        jax.ShapeDtypeStruct((NP, HID), jnp.float32),
        jax.ShapeDtypeStruct((NP, HID), jnp.float32),
        jax.ShapeDtypeStruct((NP,), jnp.float32),
    ],
    scratch_types=[
        pltpu.VMEM((CH,), jnp.int32),
        pltpu.VMEM((CH,), jnp.int32),
        pltpu.VMEM((CH,), jnp.int32),
        pltpu.VMEM((CH,), jnp.int32),
        pltpu.VMEM((CH,), jnp.float32),
        pltpu.VMEM((CH, HID), jnp.float32),
        pltpu.VMEM((CH, HID), jnp.float32),
        pltpu.VMEM((SLICE,), jnp.float32),
        pltpu.VMEM((SLICE, HID), jnp.float32),
        pltpu.VMEM_SHARED((NP,), jnp.float32),
        pltpu.VMEM_SHARED((NP, HID), jnp.float32),
        pltpu.SemaphoreType.DMA,
        pltpu.SemaphoreType.DMA,
        pltpu.SemaphoreType.DMA,
        pltpu.SemaphoreType.DMA,
    ],
)
def _mega_sc(xenc_hbm, ei_hbm,
             s_hbm, y0_hbm, y1_hbm, dinv_hbm,
             ra_v, rb_v, ca_v, cb_v, ones_v, rows0_v, rows1_v,
             dbuf_v, ybuf_v, sh_deg, sh_s,
             sem_a0, sem_a1, sem_g0, sem_g1):
    c = lax.axis_index("c")
    s = lax.axis_index("s")
    wid = s * NC + c
    cbufs = [ca_v, cb_v]
    rbufs = [ra_v, rb_v]
    rowbufs = [rows0_v, rows1_v]
    sems_a = [sem_a0, sem_a1]
    sems_g = [sem_g0, sem_g1]
    _fill_1d(ones_v, CH, 1.0)
    _fill_1d(dbuf_v, SLICE, 0.0)

    def zbody(i, carry):
        ybuf_v[i, pl.ds(0, 16)] = jnp.zeros((16,), jnp.float32)
        ybuf_v[i, pl.ds(16, 16)] = jnp.zeros((16,), jnp.float32)
        return carry
    lax.fori_loop(0, SLICE, zbody, 0)
    pltpu.sync_copy(dbuf_v, sh_deg.at[pl.ds(s * SLICE, SLICE)])
    pltpu.sync_copy(ybuf_v, sh_s.at[pl.ds(s * SLICE, SLICE), :])
    plsc.subcore_barrier()
    abase = s * (E // NS)
    h = [None, None]
    h[0] = pltpu.async_copy(ei_hbm.at[1, pl.ds(abase, CH)], cbufs[0], sems_a[0])
    for k in range(DEG_CH):
        b = k % 2
        if k + 1 < DEG_CH:
            h[1 - b] = pltpu.async_copy(
                ei_hbm.at[1, pl.ds(abase + (k + 1) * CH, CH)],
                cbufs[1 - b], sems_a[1 - b])
        h[b].wait()
        pltpu.sync_copy(ones_v, sh_deg.at[cbufs[b]], add=True)
    plsc.subcore_barrier()
    pltpu.sync_copy(sh_deg.at[pl.ds(s * SLICE, SLICE)], dbuf_v)

    def nr(i, carry):
        d = dbuf_v[pl.ds(i * 16, 16)] + 1.0
        xi = plsc.bitcast(d, jnp.int32)
        xi = jnp.full((16,), 0x5F3759DF, jnp.int32) \
            - lax.shift_right_logical(xi, jnp.ones((16,), jnp.int32))
        xx = plsc.bitcast(xi, jnp.float32)
        for _ in range(4):
            xx = xx * (1.5 - 0.5 * d * xx * xx)
        dbuf_v[pl.ds(i * 16, 16)] = xx
        return carry
    lax.fori_loop(0, SLICE // 16, nr, 0)
    @pl.when(c == 0)
    def _():
        pltpu.sync_copy(dbuf_v, dinv_hbm.at[pl.ds(s * SLICE, SLICE)])
    pltpu.sync_copy(xenc_hbm.at[pl.ds(s * SLICE, SLICE), :], ybuf_v)

    def scale_row(i, carry):
        splat = jnp.zeros((16,), jnp.int32) + i
        dv = plsc.load_gather(dbuf_v, [splat])
        ybuf_v[i, pl.ds(0, 16)] = ybuf_v[i, pl.ds(0, 16)] * dv
        ybuf_v[i, pl.ds(16, 16)] = ybuf_v[i, pl.ds(16, 16)] * dv
        return carry
    lax.fori_loop(0, SLICE, scale_row, 0)

    @pl.when(c == 0)
    def _():
        pltpu.sync_copy(ybuf_v, y0_hbm.at[pl.ds(s * SLICE, SLICE), :])

    @pl.when(c == 1)
    def _():
        pltpu.sync_copy(ybuf_v, y1_hbm.at[pl.ds(s * SLICE, SLICE), :])
    plsc.subcore_barrier()
    cbase = wid * E_PER_W

    def phase_c(y_hbm):
        hg = [None, None]
        pltpu.sync_copy(ei_hbm.at[0, pl.ds(cbase, CH)], rbufs[0])
        pltpu.sync_copy(ei_hbm.at[1, pl.ds(cbase, CH)], cbufs[0])
        hg[0] = pltpu.async_copy(y_hbm.at[rbufs[0]], rowbufs[0], sems_g[0])
        for k in range(NCH_C):
            b = k % 2
            if k + 1 < NCH_C:
                pltpu.sync_copy(
                    ei_hbm.at[0, pl.ds(cbase + (k + 1) * CH, CH)],
                    rbufs[1 - b])
                pltpu.sync_copy(
                    ei_hbm.at[1, pl.ds(cbase + (k + 1) * CH, CH)],
                    cbufs[1 - b])
                hg[1 - b] = pltpu.async_copy(y_hbm.at[rbufs[1 - b]],
                                             rowbufs[1 - b], sems_g[1 - b])
            hg[b].wait()
            pltpu.sync_copy(rowbufs[b], sh_s.at[cbufs[b]], add=True)

    @pl.when(c == 0)
    def _():
        phase_c(y0_hbm)

    @pl.when(c == 1)
    def _():
        phase_c(y1_hbm)
    plsc.subcore_barrier()
    pltpu.sync_copy(sh_s.at[pl.ds(s * SLICE, SLICE), :], ybuf_v)
    pltpu.sync_copy(ybuf_v, s_hbm.at[c, pl.ds(s * SLICE, SLICE), :])


@functools.partial(
    pl.kernel,
    mesh=_mesh,
    compiler_params=pltpu.CompilerParams(use_tc_tiling_on_sc=False,
                                         needs_layout_passes=False),
    out_type=jax.ShapeDtypeStruct((E,), jnp.float32),
    scratch_types=[
        pltpu.VMEM((NP,), jnp.float32),
        pltpu.VMEM((NP,), jnp.float32),
        pltpu.VMEM((CHUNK,), jnp.int32),
        pltpu.VMEM((CHUNK,), jnp.int32),
        pltpu.VMEM((CHUNK,), jnp.int32),
        pltpu.VMEM((CHUNK,), jnp.int32),
        pltpu.VMEM((CHUNK,), jnp.float32),
        pltpu.VMEM((CHUNK,), jnp.float32),
        pltpu.VMEM((CHUNK,), jnp.float32),
        pltpu.VMEM((CHUNK,), jnp.float32),
        pltpu.SemaphoreType.DMA,
        pltpu.SemaphoreType.DMA,
        pltpu.SemaphoreType.DMA,
        pltpu.SemaphoreType.DMA,
        pltpu.SemaphoreType.DMA,
        pltpu.SemaphoreType.DMA,
        pltpu.SemaphoreType.DMA,
        pltpu.SemaphoreType.DMA,
        pltpu.SemaphoreType.DMA,
        pltpu.SemaphoreType.DMA,
    ],
)
def _edgeout_sc(u_hbm, v_hbm, eb_hbm, ei_hbm, out_hbm,
                u_v, v_v, r0_v, r1_v, c0_v, c1_v, e0_v, e1_v, o0_v, o1_v,
                sem_u, sem_v, sem_r0, sem_r1, sem_c0, sem_c1, sem_e0,
                sem_e1, sem_o0, sem_o1):
    c = lax.axis_index("c")
    s = lax.axis_index("s")
    wid = s * NC + c
    rbufs, cbufs, ebufs, obufs = [r0_v, r1_v], [c0_v, c1_v], \
        [e0_v, e1_v], [o0_v, o1_v]
    sems_r, sems_c, sems_e, sems_o = [sem_r0, sem_r1], [sem_c0, sem_c1], \
        [sem_e0, sem_e1], [sem_o0, sem_o1]
    hu = pltpu.async_copy(u_hbm, u_v, sem_u)
    hv = pltpu.async_copy(v_hbm, v_v, sem_v)

    def pref(k, b):
        base = wid * E_PER_W + k * CHUNK
        return (
            pltpu.async_copy(ei_hbm.at[0, pl.ds(base, CHUNK)], rbufs[b],
                             sems_r[b]),
            pltpu.async_copy(ei_hbm.at[1, pl.ds(base, CHUNK)], cbufs[b],
                             sems_c[b]),
            pltpu.async_copy(eb_hbm.at[pl.ds(base, CHUNK)], ebufs[b],
                             sems_e[b]),
        )

    hin = [None, None]
    hout = [None, None]
    hin[0] = pref(0, 0)
    hu.wait()
    hv.wait()
    for k in range(NCH):
        b = k % 2
        if k + 1 < NCH:
            hin[1 - b] = pref(k + 1, 1 - b)
        for hh in hin[b]:
            hh.wait()
        if hout[b] is not None:
            hout[b].wait()
        ridx_v, cidx_v, eb_v, o_v = rbufs[b], cbufs[b], ebufs[b], obufs[b]

        def body(j, carry):
            r = ridx_v[pl.ds(j * 16, 16)]
            cc = cidx_v[pl.ds(j * 16, 16)]
            g = (plsc.load_gather(u_v, [r])
                 + plsc.load_gather(v_v, [cc])
                 + eb_v[pl.ds(j * 16, 16)])
            o_v[pl.ds(j * 16, 16)] = g
            return carry
        lax.fori_loop(0, CHUNK // 16, body, 0)
        base = wid * E_PER_W + k * CHUNK
        hout[b] = pltpu.async_copy(o_v, out_hbm.at[pl.ds(base, CHUNK)],
                                   sems_o[b])
    for hh in hout:
        if hh is not None:
            hh.wait()


_BE = 6400


def _pre_body(x_ref, wne_ref, bne_ref, eat_ref, wee_ref, bee_ref, wout_ref,
              bout_ref, xenc_ref, eb_ref):
    @pl.when(pl.program_id(0) == 0)
    def _():
        xw = jnp.dot(x_ref[...], wne_ref[...],
                     preferred_element_type=jnp.float32)
        xenc_ref[...] = jnp.maximum(xw + bne_ref[...], 0.0)

    tt = lax.dot_general(wee_ref[...], eat_ref[...],
                         (((0,), (0,)), ((), ())),
                         preferred_element_type=jnp.float32)
    tt = jnp.maximum(tt + bee_ref[...], 0.0)
    w3 = wout_ref[2 * HID:3 * HID, :]
    st = lax.dot_general(w3, tt, (((0,), (0,)), ((), ())),
                         preferred_element_type=jnp.float32) + bout_ref[...]
    i = pl.program_id(0)
    eb_ref[pl.ds(i * _BE, _BE)] = jnp.reshape(st, (_BE,))


def _pre_tc(x_pad, W_ne, b_ne2, ea_t, W_ee, b_ee_col, W_out, b_out2):
    return pl.pallas_call(
        _pre_body,
        grid=(E // _BE,),
        in_specs=[
            pl.BlockSpec((NP, D_IN), lambda i: (0, 0)),
            pl.BlockSpec((D_IN, HID), lambda i: (0, 0)),
            pl.BlockSpec((1, HID), lambda i: (0, 0)),
            pl.BlockSpec((D_EDGE, _BE), lambda i: (0, i)),
            pl.BlockSpec((D_EDGE, HID), lambda i: (0, 0)),
            pl.BlockSpec((HID, 1), lambda i: (0, 0)),
            pl.BlockSpec((3 * HID, 1), lambda i: (0, 0)),
            pl.BlockSpec((1, 1), lambda i: (0, 0)),
        ],
        out_specs=[
            pl.BlockSpec((NP, HID), lambda i: (0, 0)),
            pl.BlockSpec((E,), lambda i: (0,)),
        ],
        out_shape=[
            jax.ShapeDtypeStruct((NP, HID), jnp.float32),
            jax.ShapeDtypeStruct((E,), jnp.float32),
        ],
    )(x_pad, W_ne, b_ne2, ea_t, W_ee, b_ee_col, W_out, b_out2)


def _huv_body(s_ref, y_ref, dinv_ref, wz_ref, lzw_ref, lzb_ref, bz_ref,
              wh_ref, lhw_ref, lhb_ref, bh_ref, wout_ref, u_ref, v_ref):
    agg = (s_ref[0] + s_ref[1] + y_ref[...]) * dinv_ref[...]
    lzw = lzw_ref[0:HID, :]
    lhw = lhw_ref[0:HID, :]
    mz = jnp.dot(wz_ref[...], lzw, preferred_element_type=jnp.float32)
    cz = jnp.dot(bz_ref[...], lzw, preferred_element_type=jnp.float32) \
        + lzb_ref[...]
    mh = jnp.dot(wh_ref[...], lhw, preferred_element_type=jnp.float32)
    ch = jnp.dot(bh_ref[...], lhw, preferred_element_type=jnp.float32) \
        + lhb_ref[...]
    z = jax.nn.sigmoid(
        jnp.dot(agg, mz, preferred_element_type=jnp.float32) + cz)
    ht = jnp.tanh(jnp.dot(agg, mh, preferred_element_type=jnp.float32) + ch)
    h = (1.0 - z) * ht
    u_ref[...] = jnp.reshape(
        jnp.dot(h, wout_ref[0:HID, :], preferred_element_type=jnp.float32),
        (NP,))
    v_ref[...] = jnp.reshape(
        jnp.dot(h, wout_ref[HID:2 * HID, :],
                preferred_element_type=jnp.float32), (NP,))


def _huv_tc(S, y, dinv, Wz, LzW, Lzb2, bz2, Wh, LhW, Lhb2, bh2, W_out):
    return pl.pallas_call(
        _huv_body,
        out_shape=[
            jax.ShapeDtypeStruct((NP,), jnp.float32),
            jax.ShapeDtypeStruct((NP,), jnp.float32),
        ],
    )(S, y, dinv, Wz, LzW, Lzb2, bz2, Wh, LhW, Lhb2, bh2, W_out)


def kernel(x, edge_index, edge_attr, W_ne, b_ne, W_ee, b_ee, Wz, bz, LzW,
           Lzb, Wr, br, LrW, Lrb, Wh, bh, LhW, Lhb, W_out, b_out):
    ei = edge_index.astype(jnp.int32)
    x_pad = jnp.pad(x, ((0, NP - N), (0, 0)))

    x_enc, eb = _pre_tc(x_pad, W_ne, b_ne.reshape(1, HID), edge_attr.T,
                        W_ee, b_ee.reshape(HID, 1), W_out,
                        b_out.reshape(1, 1))
    S, y0, _y1, dinv = _mega_sc(x_enc, ei)
    u1, v1 = _huv_tc(S, y0, dinv.reshape(NP, 1), Wz, LzW,
                     Lzb.reshape(1, HID), bz.reshape(1, HID), Wh, LhW,
                     Lhb.reshape(1, HID), bh.reshape(1, HID), W_out)
    out = _edgeout_sc(u1, v1, eb, ei)
    return out.reshape(E, 1)


# speedup vs baseline: 2.8325x; 1.2724x over previous
import functools

import jax
import jax.numpy as jnp
from jax import lax
from jax.experimental import pallas as pl
from jax.experimental.pallas import tpu as pltpu
from jax.experimental.pallas import tpu_sc as plsc

N = 10000
NP = 10240
E = 320000
D_IN = 128
D_EDGE = 16
HID = 32

NC = 2
NS = 16
NW = NC * NS
E_PER_W = E // NW
CHUNK = 2000
NCH = E_PER_W // CHUNK
SLICE = NP // NS

_mesh = plsc.VectorSubcoreMesh(core_axis_name="c", subcore_axis_name="s")


def _fill_1d(buf, n, val):
    def body(i, carry):
        buf[pl.ds(i * 16, 16)] = jnp.full((16,), val, jnp.float32)
        return carry
    lax.fori_loop(0, n // 16, body, 0)


CH = 1000
NCH_C = E_PER_W // CH
DEG_CH = (E // NS) // CH


@functools.partial(
    pl.kernel,
    mesh=_mesh,
    compiler_params=pltpu.CompilerParams(use_tc_tiling_on_sc=False,
                                         needs_layout_passes=False),
    out_type=[
        jax.ShapeDtypeStruct((NC, NP, HID), jnp.float32),
        jax.ShapeDtypeStruct((NP, HID), jnp.float32),
        jax.ShapeDtypeStruct((NP, HID), jnp.float32),
        jax.ShapeDtypeStruct((NP,), jnp.float32),
    ],
    scratch_types=[
        pltpu.VMEM((CH,), jnp.int32),
        pltpu.VMEM((CH,), jnp.int32),
        pltpu.VMEM((CH,), jnp.int32),
        pltpu.VMEM((CH,), jnp.int32),
        pltpu.VMEM((CH,), jnp.float32),
        pltpu.VMEM((CH, HID), jnp.float32),
        pltpu.VMEM((CH, HID), jnp.float32),
        pltpu.VMEM((SLICE,), jnp.float32),
        pltpu.VMEM((SLICE, HID), jnp.float32),
        pltpu.VMEM_SHARED((NP,), jnp.float32),
        pltpu.VMEM_SHARED((NP, HID), jnp.float32),
        pltpu.SemaphoreType.DMA,
        pltpu.SemaphoreType.DMA,
        pltpu.SemaphoreType.DMA,
        pltpu.SemaphoreType.DMA,
    ],
)
def _mega_sc(xenc_hbm, ei_hbm,
             s_hbm, y0_hbm, y1_hbm, dinv_hbm,
             ra_v, rb_v, ca_v, cb_v, ones_v, rows0_v, rows1_v,
             dbuf_v, ybuf_v, sh_deg, sh_s,
             sem_a0, sem_a1, sem_g0, sem_g1):
    c = lax.axis_index("c")
    s = lax.axis_index("s")
    wid = s * NC + c
    cbufs = [ca_v, cb_v]
    rbufs = [ra_v, rb_v]
    rowbufs = [rows0_v, rows1_v]
    sems_a = [sem_a0, sem_a1]
    sems_g = [sem_g0, sem_g1]
    _fill_1d(ones_v, CH, 1.0)
    _fill_1d(dbuf_v, SLICE, 0.0)

    def zbody(i, carry):
        ybuf_v[i, pl.ds(0, 16)] = jnp.zeros((16,), jnp.float32)
        ybuf_v[i, pl.ds(16, 16)] = jnp.zeros((16,), jnp.float32)
        return carry
    lax.fori_loop(0, SLICE, zbody, 0)
    pltpu.sync_copy(dbuf_v, sh_deg.at[pl.ds(s * SLICE, SLICE)])
    pltpu.sync_copy(ybuf_v, sh_s.at[pl.ds(s * SLICE, SLICE), :])
    plsc.subcore_barrier()
    abase = s * (E // NS)
    h = [None, None]
    h[0] = pltpu.async_copy(ei_hbm.at[1, pl.ds(abase, CH)], cbufs[0], sems_a[0])
    for k in range(DEG_CH):
        b = k % 2
        if k + 1 < DEG_CH:
            h[1 - b] = pltpu.async_copy(
                ei_hbm.at[1, pl.ds(abase + (k + 1) * CH, CH)],
                cbufs[1 - b], sems_a[1 - b])
        h[b].wait()
        pltpu.sync_copy(ones_v, sh_deg.at[cbufs[b]], add=True)
    plsc.subcore_barrier()
    pltpu.sync_copy(sh_deg.at[pl.ds(s * SLICE, SLICE)], dbuf_v)

    def nr(i, carry):
        d = dbuf_v[pl.ds(i * 16, 16)] + 1.0
        xi = plsc.bitcast(d, jnp.int32)
        xi = jnp.full((16,), 0x5F3759DF, jnp.int32) \
            - lax.shift_right_logical(xi, jnp.ones((16,), jnp.int32))
        xx = plsc.bitcast(xi, jnp.float32)
        for _ in range(4):
            xx = xx * (1.5 - 0.5 * d * xx * xx)
        dbuf_v[pl.ds(i * 16, 16)] = xx
        return carry
    lax.fori_loop(0, SLICE // 16, nr, 0)
    @pl.when(c == 0)
    def _():
        pltpu.sync_copy(dbuf_v, dinv_hbm.at[pl.ds(s * SLICE, SLICE)])
    pltpu.sync_copy(xenc_hbm.at[pl.ds(s * SLICE, SLICE), :], ybuf_v)

    def scale_row(i, carry):
        splat = jnp.zeros((16,), jnp.int32) + i
        dv = plsc.load_gather(dbuf_v, [splat])
        ybuf_v[i, pl.ds(0, 16)] = ybuf_v[i, pl.ds(0, 16)] * dv
        ybuf_v[i, pl.ds(16, 16)] = ybuf_v[i, pl.ds(16, 16)] * dv
        return carry
    lax.fori_loop(0, SLICE, scale_row, 0)

    @pl.when(c == 0)
    def _():
        pltpu.sync_copy(ybuf_v, y0_hbm.at[pl.ds(s * SLICE, SLICE), :])

    @pl.when(c == 1)
    def _():
        pltpu.sync_copy(ybuf_v, y1_hbm.at[pl.ds(s * SLICE, SLICE), :])
    plsc.subcore_barrier()
    cbase = wid * E_PER_W

    def phase_c(y_hbm):
        hg = [None, None]
        pltpu.sync_copy(ei_hbm.at[0, pl.ds(cbase, CH)], rbufs[0])
        pltpu.sync_copy(ei_hbm.at[1, pl.ds(cbase, CH)], cbufs[0])
        hg[0] = pltpu.async_copy(y_hbm.at[rbufs[0]], rowbufs[0], sems_g[0])
        for k in range(NCH_C):
            b = k % 2
            if k + 1 < NCH_C:
                pltpu.sync_copy(
                    ei_hbm.at[0, pl.ds(cbase + (k + 1) * CH, CH)],
                    rbufs[1 - b])
                pltpu.sync_copy(
                    ei_hbm.at[1, pl.ds(cbase + (k + 1) * CH, CH)],
                    cbufs[1 - b])
                hg[1 - b] = pltpu.async_copy(y_hbm.at[rbufs[1 - b]],
                                             rowbufs[1 - b], sems_g[1 - b])
            hg[b].wait()
            pltpu.sync_copy(rowbufs[b], sh_s.at[cbufs[b]], add=True)

    @pl.when(c == 0)
    def _():
        phase_c(y0_hbm)

    @pl.when(c == 1)
    def _():
        phase_c(y1_hbm)
    plsc.subcore_barrier()
    pltpu.sync_copy(sh_s.at[pl.ds(s * SLICE, SLICE), :], ybuf_v)
    pltpu.sync_copy(ybuf_v, s_hbm.at[c, pl.ds(s * SLICE, SLICE), :])


@functools.partial(
    pl.kernel,
    mesh=_mesh,
    compiler_params=pltpu.CompilerParams(use_tc_tiling_on_sc=False,
                                         needs_layout_passes=False),
    out_type=jax.ShapeDtypeStruct((E,), jnp.float32),
    scratch_types=[
        pltpu.VMEM((NP,), jnp.float32),
        pltpu.VMEM((NP,), jnp.float32),
        pltpu.VMEM((CHUNK,), jnp.int32),
        pltpu.VMEM((CHUNK,), jnp.int32),
        pltpu.VMEM((CHUNK,), jnp.int32),
        pltpu.VMEM((CHUNK,), jnp.int32),
        pltpu.VMEM((CHUNK,), jnp.float32),
        pltpu.VMEM((CHUNK,), jnp.float32),
        pltpu.VMEM((CHUNK,), jnp.float32),
        pltpu.VMEM((CHUNK,), jnp.float32),
        pltpu.SemaphoreType.DMA,
        pltpu.SemaphoreType.DMA,
        pltpu.SemaphoreType.DMA,
        pltpu.SemaphoreType.DMA,
        pltpu.SemaphoreType.DMA,
        pltpu.SemaphoreType.DMA,
        pltpu.SemaphoreType.DMA,
        pltpu.SemaphoreType.DMA,
        pltpu.SemaphoreType.DMA,
        pltpu.SemaphoreType.DMA,
    ],
)
def _edgeout_sc(u_hbm, v_hbm, eb_hbm, ei_hbm, out_hbm,
                u_v, v_v, r0_v, r1_v, c0_v, c1_v, e0_v, e1_v, o0_v, o1_v,
                sem_u, sem_v, sem_r0, sem_r1, sem_c0, sem_c1, sem_e0,
                sem_e1, sem_o0, sem_o1):
    c = lax.axis_index("c")
    s = lax.axis_index("s")
    wid = s * NC + c
    rbufs, cbufs, ebufs, obufs = [r0_v, r1_v], [c0_v, c1_v], \
        [e0_v, e1_v], [o0_v, o1_v]
    sems_r, sems_c, sems_e, sems_o = [sem_r0, sem_r1], [sem_c0, sem_c1], \
        [sem_e0, sem_e1], [sem_o0, sem_o1]
    hu = pltpu.async_copy(u_hbm, u_v, sem_u)
    hv = pltpu.async_copy(v_hbm, v_v, sem_v)

    def pref(k, b):
        base = wid * E_PER_W + k * CHUNK
        return (
            pltpu.async_copy(ei_hbm.at[0, pl.ds(base, CHUNK)], rbufs[b],
                             sems_r[b]),
            pltpu.async_copy(ei_hbm.at[1, pl.ds(base, CHUNK)], cbufs[b],
                             sems_c[b]),
            pltpu.async_copy(eb_hbm.at[pl.ds(base, CHUNK)], ebufs[b],
                             sems_e[b]),
        )

    hin = [None, None]
    hout = [None, None]
    hin[0] = pref(0, 0)
    hu.wait()
    hv.wait()
    for k in range(NCH):
        b = k % 2
        if k + 1 < NCH:
            hin[1 - b] = pref(k + 1, 1 - b)
        for hh in hin[b]:
            hh.wait()
        if hout[b] is not None:
            hout[b].wait()
        ridx_v, cidx_v, eb_v, o_v = rbufs[b], cbufs[b], ebufs[b], obufs[b]

        def body(j, carry):
            r = ridx_v[pl.ds(j * 16, 16)]
            cc = cidx_v[pl.ds(j * 16, 16)]
            g = (plsc.load_gather(u_v, [r])
                 + plsc.load_gather(v_v, [cc])
                 + eb_v[pl.ds(j * 16, 16)])
            o_v[pl.ds(j * 16, 16)] = g
            return carry
        lax.fori_loop(0, CHUNK // 16, body, 0)
        base = wid * E_PER_W + k * CHUNK
        hout[b] = pltpu.async_copy(o_v, out_hbm.at[pl.ds(base, CHUNK)],
                                   sems_o[b])
    for hh in hout:
        if hh is not None:
            hh.wait()


_BE = 6400


def _xenc_body(x_ref, wne_ref, bne_ref, xenc_ref):
    xw = jnp.dot(x_ref[...], wne_ref[...], preferred_element_type=jnp.float32)
    xenc_ref[...] = jnp.maximum(xw + bne_ref[...], 0.0)


def _xenc_tc(x_pad, W_ne, b_ne2):
    return pl.pallas_call(
        _xenc_body,
        out_shape=jax.ShapeDtypeStruct((NP, HID), jnp.float32),
    )(x_pad, W_ne, b_ne2)


def _eb_body(eat_ref, wee_ref, bee_ref, wout_ref, bout_ref, eb_ref):
    tt = lax.dot_general(wee_ref[...], eat_ref[...],
                         (((0,), (0,)), ((), ())),
                         preferred_element_type=jnp.float32)
    tt = jnp.maximum(tt + bee_ref[...], 0.0)
    w3 = wout_ref[2 * HID:3 * HID, :]
    st = lax.dot_general(w3, tt, (((0,), (0,)), ((), ())),
                         preferred_element_type=jnp.float32) + bout_ref[...]
    i = pl.program_id(0)
    eb_ref[pl.ds(i * _BE, _BE)] = jnp.reshape(st, (_BE,))


def _eb_tc(ea_t, W_ee, b_ee_col, W_out, b_out2):
    return pl.pallas_call(
        _eb_body,
        grid=(E // _BE,),
        in_specs=[
            pl.BlockSpec((D_EDGE, _BE), lambda i: (0, i)),
            pl.BlockSpec((D_EDGE, HID), lambda i: (0, 0)),
            pl.BlockSpec((HID, 1), lambda i: (0, 0)),
            pl.BlockSpec((3 * HID, 1), lambda i: (0, 0)),
            pl.BlockSpec((1, 1), lambda i: (0, 0)),
        ],
        out_specs=pl.BlockSpec((E,), lambda i: (0,)),
        out_shape=jax.ShapeDtypeStruct((E,), jnp.float32),
    )(ea_t, W_ee, b_ee_col, W_out, b_out2)


def _huv_body(s_ref, y_ref, dinv_ref, wz_ref, lzw_ref, lzb_ref, bz_ref,
              wh_ref, lhw_ref, lhb_ref, bh_ref, wout_ref, u_ref, v_ref):
    agg = (s_ref[0] + s_ref[1] + y_ref[...]) * dinv_ref[...]
    lzw = lzw_ref[0:HID, :]
    lhw = lhw_ref[0:HID, :]
    mz = jnp.dot(wz_ref[...], lzw, preferred_element_type=jnp.float32)
    cz = jnp.dot(bz_ref[...], lzw, preferred_element_type=jnp.float32) \
        + lzb_ref[...]
    mh = jnp.dot(wh_ref[...], lhw, preferred_element_type=jnp.float32)
    ch = jnp.dot(bh_ref[...], lhw, preferred_element_type=jnp.float32) \
        + lhb_ref[...]
    z = jax.nn.sigmoid(
        jnp.dot(agg, mz, preferred_element_type=jnp.float32) + cz)
    ht = jnp.tanh(jnp.dot(agg, mh, preferred_element_type=jnp.float32) + ch)
    h = (1.0 - z) * ht
    u_ref[...] = jnp.reshape(
        jnp.dot(h, wout_ref[0:HID, :], preferred_element_type=jnp.float32),
        (NP,))
    v_ref[...] = jnp.reshape(
        jnp.dot(h, wout_ref[HID:2 * HID, :],
                preferred_element_type=jnp.float32), (NP,))


def _huv_tc(S, y, dinv, Wz, LzW, Lzb2, bz2, Wh, LhW, Lhb2, bh2, W_out):
    return pl.pallas_call(
        _huv_body,
        out_shape=[
            jax.ShapeDtypeStruct((NP,), jnp.float32),
            jax.ShapeDtypeStruct((NP,), jnp.float32),
        ],
    )(S, y, dinv, Wz, LzW, Lzb2, bz2, Wh, LhW, Lhb2, bh2, W_out)


def kernel(x, edge_index, edge_attr, W_ne, b_ne, W_ee, b_ee, Wz, bz, LzW,
           Lzb, Wr, br, LrW, Lrb, Wh, bh, LhW, Lhb, W_out, b_out):
    ei = edge_index.astype(jnp.int32)
    x_pad = jnp.pad(x, ((0, NP - N), (0, 0)))

    x_enc = _xenc_tc(x_pad, W_ne, b_ne.reshape(1, HID))
    S, y0, _y1, dinv = _mega_sc(x_enc, ei)
    eb = _eb_tc(edge_attr.T, W_ee, b_ee.reshape(HID, 1), W_out,
                b_out.reshape(1, 1))
    u1, v1 = _huv_tc(S, y0, dinv.reshape(NP, 1), Wz, LzW,
                     Lzb.reshape(1, HID), bz.reshape(1, HID), Wh, LhW,
                     Lhb.reshape(1, HID), bh.reshape(1, HID), W_out)
    out = _edgeout_sc(u1, v1, eb, ei)
    return out.reshape(E, 1)
